# baseline (speedup 1.0000x reference)
import functools

import jax
import jax.numpy as jnp
from jax import lax
from jax.experimental import pallas as pl
from jax.experimental.pallas import tpu as pltpu
from jax.experimental.pallas import tpu_sc as plsc

N_NODES = 10000
N_EDGES = 320000
FDIM = 16
W = 128
NC = 2
NS = 16
EDGES_PER_CORE = N_EDGES // NC
EDGES_PER_TILE = EDGES_PER_CORE // NS
CH = 160
CHS = 40
NSUB = CH // CHS
NL = 6
NSC = 2
N_CHUNKS_F = EDGES_PER_TILE // CH
N_TAIL_EDGES = EDGES_PER_TILE - N_CHUNKS_F * CH
N_CHUNKS = N_CHUNKS_F + 1
N_PAD = 10240
ROWS_PER_TILE = N_PAD // NS


def _sc_scatter_partials(dst, edge_attr):
    mesh = plsc.VectorSubcoreMesh(core_axis_name="c", subcore_axis_name="s")
    out_type = jax.ShapeDtypeStruct((NC, N_PAD, W), jnp.float32)

    scratch = [
        pltpu.VMEM_SHARED((N_PAD, W), jnp.float32),
        pltpu.VMEM((NL * CH,), jnp.int32),
        pltpu.VMEM((NL * CH * FDIM,), jnp.float32),
        pltpu.VMEM((CHS,), jnp.int32),
        pltpu.VMEM((CHS,), jnp.int32),
        pltpu.VMEM((CHS, W), jnp.float32),
        pltpu.VMEM((CHS, W), jnp.float32),
        pltpu.SemaphoreType.DMA((NL,)),
        pltpu.SemaphoreType.DMA,
        pltpu.SemaphoreType.DMA,
    ]

    @functools.partial(pl.kernel, out_type=out_type, mesh=mesh,
                       scratch_types=scratch)
    def scatter_kernel(dst_hbm, attr_hbm, pacc_hbm, acc_sh,
                       idx_all, attr_all, sidx0, sidx1, stage0, stage1,
                       sem_load, sem_s0, sem_s1):
        sidx_v = (sidx0, sidx1)
        stage_v = (stage0, stage1)
        sem_scat = (sem_s0, sem_s1)

        c = lax.axis_index("c")
        s = lax.axis_index("s")

        zv = jnp.zeros((16,), jnp.float32)

        @pl.loop(0, CHS)
        def _(e):
            @pl.loop(0, W // 16)
            def _(j):
                stage_v[0][e, pl.ds(j * 16, 16)] = zv

        row0 = s * ROWS_PER_TILE

        @pl.loop(0, ROWS_PER_TILE // CHS)
        def _(b):
            pltpu.sync_copy(stage_v[0], acc_sh.at[pl.ds(row0 + b * CHS, CHS)])

        lane = lax.iota(jnp.int32, 16)
        onehot = jnp.where(lane == 0, 1.0, 0.0).astype(jnp.float32)
        for b in range(NSC):
            @pl.loop(0, CHS)
            def _(e, b=b):
                stage_v[b][e, pl.ds(16, 16)] = onehot
                if b != 0:
                    @pl.loop(2, W // 16)
                    def _(j):
                        stage_v[b][e, pl.ds(j * 16, 16)] = zv

        plsc.subcore_barrier()

        wid = c * NS + s
        base = wid * EDGES_PER_TILE

        def fire_loads(k, slot):
            off = base + k * CH
            loc = slot * CH
            pltpu.async_copy(dst_hbm.at[pl.ds(off, CH)],
                             idx_all.at[pl.ds(loc, CH)], sem_load.at[slot])
            pltpu.async_copy(attr_hbm.at[pl.ds(off * FDIM, CH * FDIM)],
                             attr_all.at[pl.ds(loc * FDIM, CH * FDIM)],
                             sem_load.at[slot])

        def wait_loads(slot):
            loc = slot * CH
            pltpu.make_async_copy(dst_hbm.at[pl.ds(0, CH)],
                                  idx_all.at[pl.ds(loc, CH)],
                                  sem_load.at[slot]).wait()
            pltpu.make_async_copy(attr_hbm.at[pl.ds(0, CH * FDIM)],
                                  attr_all.at[pl.ds(loc * FDIM, CH * FDIM)],
                                  sem_load.at[slot]).wait()

        def wait_scatter(h):
            pltpu.make_async_copy(stage_v[h], acc_sh.at[sidx_v[h]],
                                  sem_scat[h]).wait()

        @pl.loop(0, NL)
        def _(b):
            fire_loads(b, b)

        @pl.loop(0, N_CHUNKS)
        def _(k):
            slot = lax.rem(k, NL)
            loc = slot * CH
            wait_loads(slot)
            for h in range(NSUB):
                sb = h % NSC

                def sub_chunk(k=k, h=h, sb=sb, loc=loc):
                    e0 = loc + h * CHS
                    sidx_v[sb][pl.ds(0, 16)] = idx_all[pl.ds(e0, 16)]
                    sidx_v[sb][pl.ds(16, 16)] = idx_all[pl.ds(e0 + 16, 16)]
                    sidx_v[sb][pl.ds(CHS - 16, 16)] = (
                        idx_all[pl.ds(e0 + CHS - 16, 16)])

                    @pl.loop(0, CHS)
                    def _(r):
                        stage_v[sb][r, pl.ds(0, 16)] = (
                            attr_all[pl.ds((e0 + r) * FDIM, 16)])

                    pltpu.async_copy(stage_v[sb], acc_sh.at[sidx_v[sb]],
                                     sem_scat[sb], add=True)

                if h < NSC:
                    @pl.when(k > 0)
                    def _(sb=sb):
                        wait_scatter(sb)
                    sub_chunk()
                else:
                    @pl.when(k < N_CHUNKS_F)
                    def _(sub_chunk=sub_chunk, sb=sb):
                        wait_scatter(sb)
                        sub_chunk()

            @pl.when(k + NL < N_CHUNKS)
            def _():
                fire_loads(k + NL, slot)

        for h in range(NSC):
            wait_scatter(h)

        plsc.subcore_barrier()
        pltpu.sync_copy(acc_sh.at[pl.ds(row0, ROWS_PER_TILE)],
                        pacc_hbm.at[c, pl.ds(row0, ROWS_PER_TILE)])

    return scatter_kernel(dst, edge_attr)


def _divide_body(pa_ref, o_ref):
    s = pa_ref[0] + pa_ref[1]
    cnt = jnp.maximum(s[:, 16:17], 1.0)
    o_ref[...] = s / cnt


def _tc_combine_divide(pacc):
    out = pl.pallas_call(
        _divide_body,
        out_shape=jax.ShapeDtypeStruct((N_PAD, W), jnp.float32),
    )(pacc)
    return out[:N_NODES, :FDIM]


def kernel(x, edge_index, edge_attr):
    del x
    dst = edge_index[1]
    dst = jnp.concatenate([dst, jnp.zeros((N_TAIL_EDGES,), dst.dtype)])
    edge_attr = jnp.concatenate(
        [edge_attr, jnp.zeros((N_TAIL_EDGES, FDIM), edge_attr.dtype)])
    pacc = _sc_scatter_partials(dst, edge_attr.reshape(-1))
    return _tc_combine_divide(pacc)
